# baseline (speedup 1.0000x reference)
import functools

import jax
import jax.numpy as jnp
from jax import lax
from jax.experimental import pallas as pl
from jax.experimental.pallas import tpu as pltpu
from jax.experimental.pallas import tpu_sc as plsc

VOCAB = 100000
N_PREFIX = 200
EMBED_DIM = 64
BATCH = 4096
HIST = 200
PAD_DIM = 128
HIST_PAD = 256

NC = 2
NS = 16
NW = NC * NS

GA, GB = 128, HIST - 128
NBUF = 4
LEAD = 2
ROWS_PER_IT = NBUF // 2


def _slot(j, b):
    r = ROWS_PER_IT * j + (b // 2)
    h0 = 0 if b % 2 == 0 else GA
    n = GA if b % 2 == 0 else GB
    return r, h0, n


def _sc_gather(table_pad, idx_pad):
    nbatch = idx_pad.shape[0]
    ROWS_PER_W = nbatch // NW
    NITER = ROWS_PER_W // ROWS_PER_IT
    mesh = plsc.VectorSubcoreMesh(
        core_axis_name="c", subcore_axis_name="s", num_cores=NC, num_subcores=NS
    )

    @functools.partial(
        pl.kernel,
        out_type=jax.ShapeDtypeStruct((nbatch, HIST, PAD_DIM), jnp.float32),
        mesh=mesh,
        compiler_params=pltpu.CompilerParams(use_tc_tiling_on_sc=True),
        scratch_types=[
            pltpu.VMEM((ROWS_PER_W, GA), jnp.int32),
            pltpu.VMEM((ROWS_PER_W, GA), jnp.int32),
            pltpu.VMEM((NBUF, GA, PAD_DIM), jnp.float32),
        ]
        + [pltpu.SemaphoreType.DMA] * (2 * NBUF),
    )
    def body(table_hbm, idx_hbm, out_hbm, idx_a, idx_b, rows, *sems):
        gsems = sems[:NBUF]
        ssems = sems[NBUF:]
        wid = lax.axis_index("s") * NC + lax.axis_index("c")
        rbase = wid * ROWS_PER_W

        pltpu.sync_copy(idx_hbm.at[pl.ds(rbase, ROWS_PER_W), pl.ds(0, GA)], idx_a)
        pltpu.sync_copy(idx_hbm.at[pl.ds(rbase, ROWS_PER_W), pl.ds(GA, GA)], idx_b)

        def idx_vec(j, b):
            r, h0, n = _slot(j, b)
            src = idx_a if h0 == 0 else idx_b
            return src.at[r, pl.ds(0, n)]

        def start_gather(j, b):
            _, _, n = _slot(j, b)
            pltpu.async_copy(
                table_hbm.at[idx_vec(j, b)],
                rows.at[b, pl.ds(0, n)],
                gsems[b],
            )

        def wait_gather(b):
            _, _, n = _slot(0, b)
            pltpu.make_async_copy(
                table_hbm.at[idx_vec(0, b)],
                rows.at[b, pl.ds(0, n)],
                gsems[b],
            ).wait()

        def start_scatter(j, b):
            r, h0, n = _slot(j, b)
            pltpu.async_copy(
                rows.at[b, pl.ds(0, n)],
                out_hbm.at[rbase + r, pl.ds(h0, n)],
                ssems[b],
            )

        def wait_scatter(b):
            _, h0, n = _slot(0, b)
            pltpu.make_async_copy(
                rows.at[b, pl.ds(0, n)],
                out_hbm.at[rbase, pl.ds(h0, n)],
                ssems[b],
            ).wait()

        for b in range(LEAD):
            start_gather(0, b)

        def loop(j, carry):
            for b in range(NBUF):
                wait_gather(b)
                start_scatter(j, b)
                b2 = (b + LEAD) % NBUF
                if b2 >= LEAD:
                    @pl.when(j > 0)
                    def _():
                        wait_scatter(b2)

                    start_gather(j, b2)
                else:
                    @pl.when(j < NITER - 1)
                    def _():
                        wait_scatter(b2)
                        start_gather(j + 1, b2)

            return carry

        lax.fori_loop(0, NITER, loop, 0)

        for b in range(NBUF):
            wait_scatter(b)

    return body(table_pad, idx_pad)


NCHUNK = 4


@jax.jit
def kernel(input, W_embed, W_new):
    table = jnp.concatenate([W_embed, W_new], axis=0)
    table_pad = jnp.pad(table, ((0, 0), (0, PAD_DIM - EMBED_DIM)))
    idx_pad = jnp.pad(input.astype(jnp.int32), ((0, 0), (0, HIST_PAD - HIST)))
    one = W_new[0, 0] * 0.0 + 1.0
    cb = BATCH // NCHUNK
    outs = [
        _sc_gather(table_pad, idx_pad[c * cb:(c + 1) * cb])[:, :, :EMBED_DIM] * one
        for c in range(NCHUNK)
    ]
    return jnp.concatenate(outs, axis=0) if NCHUNK > 1 else outs[0]


# speedup vs baseline: 2.0903x; 2.0903x over previous
import functools

import jax
import jax.numpy as jnp
from jax import lax
from jax.experimental import pallas as pl
from jax.experimental.pallas import tpu as pltpu
from jax.experimental.pallas import tpu_sc as plsc

VOCAB = 100000
N_PREFIX = 200
EMBED_DIM = 64
BATCH = 4096
HIST = 200
PAD_DIM = 128
HIST_PAD = 256

NC = 2
NS = 16
NW = NC * NS

GA, GB = 128, HIST - 128
NBUF = 4
LEAD = 2
ROWS_PER_IT = NBUF // 2


def _slot(j, b):
    r = ROWS_PER_IT * j + (b // 2)
    h0 = 0 if b % 2 == 0 else GA
    n = GA if b % 2 == 0 else GB
    return r, h0, n


def _sc_gather(table_pad, idx_pad):
    nbatch = idx_pad.shape[0]
    ROWS_PER_W = nbatch // NW
    NITER = ROWS_PER_W // ROWS_PER_IT
    mesh = plsc.VectorSubcoreMesh(
        core_axis_name="c", subcore_axis_name="s", num_cores=NC, num_subcores=NS
    )

    @functools.partial(
        pl.kernel,
        out_type=jax.ShapeDtypeStruct((nbatch, HIST, PAD_DIM), jnp.float32),
        mesh=mesh,
        compiler_params=pltpu.CompilerParams(use_tc_tiling_on_sc=True),
        scratch_types=[
            pltpu.VMEM((ROWS_PER_W, GA), jnp.int32),
            pltpu.VMEM((ROWS_PER_W, GA), jnp.int32),
            pltpu.VMEM((NBUF, GA, PAD_DIM), jnp.float32),
        ]
        + [pltpu.SemaphoreType.DMA] * (2 * NBUF),
    )
    def body(table_hbm, idx_hbm, out_hbm, idx_a, idx_b, rows, *sems):
        gsems = sems[:NBUF]
        ssems = sems[NBUF:]
        wid = lax.axis_index("s") * NC + lax.axis_index("c")
        rbase = wid * ROWS_PER_W

        pltpu.sync_copy(idx_hbm.at[pl.ds(rbase, ROWS_PER_W), pl.ds(0, GA)], idx_a)
        pltpu.sync_copy(idx_hbm.at[pl.ds(rbase, ROWS_PER_W), pl.ds(GA, GA)], idx_b)

        def idx_vec(j, b):
            r, h0, n = _slot(j, b)
            src = idx_a if h0 == 0 else idx_b
            return src.at[r, pl.ds(0, n)]

        def start_gather(j, b):
            _, _, n = _slot(j, b)
            pltpu.async_copy(
                table_hbm.at[idx_vec(j, b)],
                rows.at[b, pl.ds(0, n)],
                gsems[b],
            )

        def wait_gather(b):
            _, _, n = _slot(0, b)
            pltpu.make_async_copy(
                table_hbm.at[idx_vec(0, b)],
                rows.at[b, pl.ds(0, n)],
                gsems[b],
            ).wait()

        def start_scatter(j, b):
            r, h0, n = _slot(j, b)
            pltpu.async_copy(
                rows.at[b, pl.ds(0, n)],
                out_hbm.at[rbase + r, pl.ds(h0, n)],
                ssems[b],
            )

        def wait_scatter(b):
            _, h0, n = _slot(0, b)
            pltpu.make_async_copy(
                rows.at[b, pl.ds(0, n)],
                out_hbm.at[rbase, pl.ds(h0, n)],
                ssems[b],
            ).wait()

        for b in range(LEAD):
            start_gather(0, b)

        def loop(j, carry):
            for b in range(NBUF):
                wait_gather(b)
                start_scatter(j, b)
                b2 = (b + LEAD) % NBUF
                if b2 >= LEAD:
                    @pl.when(j > 0)
                    def _():
                        wait_scatter(b2)

                    start_gather(j, b2)
                else:
                    @pl.when(j < NITER - 1)
                    def _():
                        wait_scatter(b2)
                        start_gather(j + 1, b2)

            return carry

        lax.fori_loop(0, NITER, loop, 0)

        for b in range(NBUF):
            wait_scatter(b)

    return body(table_pad, idx_pad)


@jax.jit
def kernel(input, W_embed, W_new):
    table = jnp.concatenate([W_embed, W_new], axis=0)
    table_pad = jnp.pad(table, ((0, 0), (0, PAD_DIM - EMBED_DIM)))
    idx_pad = jnp.pad(input.astype(jnp.int32), ((0, 0), (0, HIST_PAD - HIST)))
    out = _sc_gather(table_pad, idx_pad)
    return out[:, :, :EMBED_DIM]


# speedup vs baseline: 2.1065x; 1.0077x over previous
import functools

import jax
import jax.numpy as jnp
from jax import lax
from jax.experimental import pallas as pl
from jax.experimental.pallas import tpu as pltpu
from jax.experimental.pallas import tpu_sc as plsc

VOCAB = 100000
N_PREFIX = 200
EMBED_DIM = 64
BATCH = 4096
HIST = 200
PAD_DIM = 128
GROUP = 128

NC = 2
NS = 16
NW = NC * NS

NBUF = 4
LEAD = 2


def _sc_gather(table_pad, idx_groups):
    ngroups = idx_groups.shape[0]
    GPW = ngroups // NW
    NITER = GPW // NBUF
    mesh = plsc.VectorSubcoreMesh(
        core_axis_name="c", subcore_axis_name="s", num_cores=NC, num_subcores=NS
    )

    @functools.partial(
        pl.kernel,
        out_type=jax.ShapeDtypeStruct((ngroups * GROUP, PAD_DIM), jnp.float32),
        mesh=mesh,
        compiler_params=pltpu.CompilerParams(use_tc_tiling_on_sc=True),
        scratch_types=[
            pltpu.VMEM((GPW, GROUP), jnp.int32),
            pltpu.VMEM((NBUF, GROUP, PAD_DIM), jnp.float32),
        ]
        + [pltpu.SemaphoreType.DMA] * (2 * NBUF),
    )
    def body(table_hbm, idx_hbm, out_hbm, idx_t, rows, *sems):
        gsems = sems[:NBUF]
        ssems = sems[NBUF:]
        wid = lax.axis_index("s") * NC + lax.axis_index("c")
        gbase = wid * GPW

        pltpu.sync_copy(idx_hbm.at[pl.ds(gbase, GPW)], idx_t)

        def start_gather(g, b):
            pltpu.async_copy(
                table_hbm.at[idx_t.at[g]], rows.at[b], gsems[b]
            )

        def wait_gather(b):
            pltpu.make_async_copy(
                table_hbm.at[idx_t.at[0]], rows.at[b], gsems[b]
            ).wait()

        def start_scatter(g, b):
            pltpu.async_copy(
                rows.at[b],
                out_hbm.at[pl.ds((gbase + g) * GROUP, GROUP)],
                ssems[b],
            )

        def wait_scatter(b):
            pltpu.make_async_copy(
                rows.at[b],
                out_hbm.at[pl.ds(gbase * GROUP, GROUP)],
                ssems[b],
            ).wait()

        for b in range(LEAD):
            start_gather(b, b)

        def loop(j, carry):
            for b in range(NBUF):
                wait_gather(b)
                start_scatter(j * NBUF + b, b)
                b2 = (b + LEAD) % NBUF
                if b2 >= LEAD:
                    @pl.when(j > 0)
                    def _():
                        wait_scatter(b2)

                    start_gather(j * NBUF + b2, b2)
                else:
                    @pl.when(j < NITER - 1)
                    def _():
                        wait_scatter(b2)
                        start_gather((j + 1) * NBUF + b2, b2)

            return carry

        lax.fori_loop(0, NITER, loop, 0)

        for b in range(NBUF):
            wait_scatter(b)

    return body(table_pad, idx_groups)


@jax.jit
def kernel(input, W_embed, W_new):
    table = jnp.concatenate([W_embed, W_new], axis=0)
    table_pad = jnp.pad(table, ((0, 0), (0, PAD_DIM - EMBED_DIM)))
    idx_groups = input.astype(jnp.int32).reshape(BATCH * HIST // GROUP, GROUP)
    out = _sc_gather(table_pad, idx_groups)
    return out.reshape(BATCH, HIST, PAD_DIM)[:, :, :EMBED_DIM]


# speedup vs baseline: 2.1097x; 1.0015x over previous
import functools

import jax
import jax.numpy as jnp
from jax import lax
from jax.experimental import pallas as pl
from jax.experimental.pallas import tpu as pltpu
from jax.experimental.pallas import tpu_sc as plsc

VOCAB = 100000
N_PREFIX = 200
EMBED_DIM = 64
BATCH = 4096
HIST = 200
PAD_DIM = 128
GROUP = 128

NC = 2
NS = 16
NW = NC * NS

NBUF = 5
LEAD = 3


def _sc_gather(table_pad, idx_groups):
    ngroups = idx_groups.shape[0]
    GPW = ngroups // NW
    NITER = GPW // NBUF
    mesh = plsc.VectorSubcoreMesh(
        core_axis_name="c", subcore_axis_name="s", num_cores=NC, num_subcores=NS
    )

    @functools.partial(
        pl.kernel,
        out_type=jax.ShapeDtypeStruct((ngroups * GROUP, PAD_DIM), jnp.float32),
        mesh=mesh,
        compiler_params=pltpu.CompilerParams(use_tc_tiling_on_sc=True),
        scratch_types=[
            pltpu.VMEM((GPW, GROUP), jnp.int32),
            pltpu.VMEM((NBUF, GROUP, PAD_DIM), jnp.float32),
        ]
        + [pltpu.SemaphoreType.DMA] * (2 * NBUF),
    )
    def body(table_hbm, idx_hbm, out_hbm, idx_t, rows, *sems):
        gsems = sems[:NBUF]
        ssems = sems[NBUF:]
        wid = lax.axis_index("s") * NC + lax.axis_index("c")
        gbase = wid * GPW

        pltpu.sync_copy(idx_hbm.at[pl.ds(gbase, GPW)], idx_t)

        def start_gather(g, b):
            pltpu.async_copy(
                table_hbm.at[idx_t.at[g]], rows.at[b], gsems[b]
            )

        def wait_gather(b):
            pltpu.make_async_copy(
                table_hbm.at[idx_t.at[0]], rows.at[b], gsems[b]
            ).wait()

        def start_scatter(g, b):
            pltpu.async_copy(
                rows.at[b],
                out_hbm.at[pl.ds((gbase + g) * GROUP, GROUP)],
                ssems[b],
            )

        def wait_scatter(b):
            pltpu.make_async_copy(
                rows.at[b],
                out_hbm.at[pl.ds(gbase * GROUP, GROUP)],
                ssems[b],
            ).wait()

        for b in range(LEAD):
            start_gather(b, b)

        def loop(j, carry):
            for b in range(NBUF):
                wait_gather(b)
                start_scatter(j * NBUF + b, b)
                b2 = (b + LEAD) % NBUF
                if b2 >= LEAD:
                    @pl.when(j > 0)
                    def _():
                        wait_scatter(b2)

                    start_gather(j * NBUF + b2, b2)
                else:
                    @pl.when(j < NITER - 1)
                    def _():
                        wait_scatter(b2)
                        start_gather((j + 1) * NBUF + b2, b2)

            return carry

        lax.fori_loop(0, NITER, loop, 0)

        for b in range(NBUF):
            wait_scatter(b)

    return body(table_pad, idx_groups)


@jax.jit
def kernel(input, W_embed, W_new):
    table = jnp.concatenate([W_embed, W_new], axis=0)
    table_pad = jnp.pad(table, ((0, 0), (0, PAD_DIM - EMBED_DIM)))
    idx_groups = input.astype(jnp.int32).reshape(BATCH * HIST // GROUP, GROUP)
    out = _sc_gather(table_pad, idx_groups)
    return out.reshape(BATCH, HIST, PAD_DIM)[:, :, :EMBED_DIM]
